# baseline (speedup 1.0000x reference)
import functools
import math

import jax
import jax.numpy as jnp
from jax import lax
from jax.experimental import pallas as pl
from jax.experimental.pallas import tpu as pltpu
from jax.experimental.pallas import tpu_sc as plsc

_LANES = 16


@functools.lru_cache(maxsize=None)
def _build(batch, seq, emb):
    info = plsc.get_sparse_core_info()
    nw = info.num_cores * info.num_subcores
    nc = info.num_cores
    assert batch % nw == 0
    nb = batch // nw
    assert nb <= 128
    scale = math.sqrt(emb)

    mesh = plsc.VectorSubcoreMesh(core_axis_name="c", subcore_axis_name="s")

    @functools.partial(
        pl.kernel,
        mesh=mesh,
        out_type=jax.ShapeDtypeStruct((batch, seq, emb), jnp.float32),
        scratch_types=[
            pltpu.VMEM((seq, emb), jnp.float32),
            pltpu.VMEM((nb,), jnp.int32),
            pltpu.VMEM((nb, emb), jnp.float32),
            pltpu.SemaphoreType.DMA,
        ],
        compiler_params=pltpu.CompilerParams(use_tc_tiling_on_sc=False),
    )
    def sc_kernel(xT_hbm, table_hbm, pe_hbm, out_hbm, pe_v, idx_v, buf_v, sem):
        wid = lax.axis_index("s") * nc + lax.axis_index("c")
        b0 = wid * nb
        pltpu.sync_copy(pe_hbm.at[pl.ds(0, seq)], pe_v)

        def s_body(s, carry):
            pltpu.sync_copy(xT_hbm.at[s, pl.ds(b0, nb)], idx_v)
            pltpu.async_copy(table_hbm.at[idx_v], buf_v, sem).wait()
            pvs = [pe_v[s, pl.ds(j * _LANES, _LANES)] for j in range(emb // _LANES)]

            def row_body(r, c):
                for j, pv in enumerate(pvs):
                    sl = pl.ds(j * _LANES, _LANES)
                    buf_v[r, sl] = buf_v[r, sl] * scale + pv
                return c

            lax.fori_loop(0, nb, row_body, 0)
            pltpu.sync_copy(buf_v, out_hbm.at[pl.ds(b0, nb), s])
            return carry

        lax.fori_loop(0, seq, s_body, 0)

    return sc_kernel


def kernel(x, table, pe):
    batch, seq = x.shape
    vocab, emb = table.shape
    xT = x.T.astype(jnp.int32)
    sc = _build(batch, seq, emb)
    return sc(xT, table, pe)
